# baseline (speedup 1.0000x reference)
import jax
import jax.numpy as jnp
from jax import lax
from jax.experimental import pallas as pl
from jax.experimental.pallas import tpu as pltpu
from jax.experimental.pallas import tpu_sc as plsc

SEQ_LEN = 8192
MODEL_DIM = 1024

_info = plsc.get_sparse_core_info()
_NC, _NS = _info.num_cores, _info.num_subcores
_NW = _NC * _NS
_ROWS_PER_W = SEQ_LEN // _NW
_HALF = _ROWS_PER_W // 2
_CHUNK = 16
_NCHUNKS = _HALF // _CHUNK
_NBUF = 3


def _copy_body(table_hbm, out_hbm, bufa0, bufa1, bufa2, shared,
               al0, al1, al2, as0, as1, as2, bl0, bl1, bl2, bs0, bs1, bs2):
    sid = lax.axis_index("s")
    wid = sid * _NC + lax.axis_index("c")
    base_a = wid * _ROWS_PER_W
    base_b = base_a + _HALF
    bufs_a = (bufa0, bufa1, bufa2)
    sem_al = (al0, al1, al2)
    sem_as = (as0, as1, as2)
    sem_bl = (bl0, bl1, bl2)
    sem_bs = (bs0, bs1, bs2)

    def load_a(i):
        b = i % _NBUF
        return pltpu.make_async_copy(
            table_hbm.at[pl.ds(base_a + i * _CHUNK, _CHUNK), :],
            bufs_a[b], sem_al[b])

    def store_a(i):
        b = i % _NBUF
        return pltpu.make_async_copy(
            bufs_a[b], out_hbm.at[pl.ds(base_a + i * _CHUNK, _CHUNK), :],
            sem_as[b])

    def load_b(i):
        b = i % _NBUF
        return pltpu.make_async_copy(
            table_hbm.at[pl.ds(base_b + i * _CHUNK, _CHUNK), :],
            shared.at[sid, b], sem_bl[b])

    def store_b(i):
        b = i % _NBUF
        return pltpu.make_async_copy(
            shared.at[sid, b], out_hbm.at[pl.ds(base_b + i * _CHUNK, _CHUNK), :],
            sem_bs[b])

    for i in range(_NBUF):
        load_a(i).start()
        load_b(i).start()
    for i in range(_NCHUNKS):
        load_a(i).wait()
        store_a(i).start()
        load_b(i).wait()
        store_b(i).start()
        p = i - 1
        if p >= 0 and p + _NBUF < _NCHUNKS:
            store_a(p).wait()
            load_a(p + _NBUF).start()
            store_b(p).wait()
            load_b(p + _NBUF).start()
    _waited = [p for p in range(_NCHUNKS) if p + _NBUF < _NCHUNKS]
    for i in range(_NCHUNKS):
        if i not in _waited:
            store_a(i).wait()
            store_b(i).wait()


def kernel(x, emb_weight):
    mesh = plsc.VectorSubcoreMesh(core_axis_name="c", subcore_axis_name="s")
    copy = pl.kernel(
        _copy_body,
        mesh=mesh,
        out_type=jax.ShapeDtypeStruct((SEQ_LEN, MODEL_DIM), jnp.float32),
        scratch_types=[
            pltpu.VMEM((_CHUNK, MODEL_DIM), jnp.float32),
            pltpu.VMEM((_CHUNK, MODEL_DIM), jnp.float32),
            pltpu.VMEM((_CHUNK, MODEL_DIM), jnp.float32),
            pltpu.VMEM_SHARED((_NS, _NBUF, _CHUNK, MODEL_DIM), jnp.float32),
            pltpu.SemaphoreType.DMA,
            pltpu.SemaphoreType.DMA,
            pltpu.SemaphoreType.DMA,
            pltpu.SemaphoreType.DMA,
            pltpu.SemaphoreType.DMA,
            pltpu.SemaphoreType.DMA,
            pltpu.SemaphoreType.DMA,
            pltpu.SemaphoreType.DMA,
            pltpu.SemaphoreType.DMA,
            pltpu.SemaphoreType.DMA,
            pltpu.SemaphoreType.DMA,
            pltpu.SemaphoreType.DMA,
        ],
    )
    return copy(emb_weight)


# speedup vs baseline: 1.0183x; 1.0183x over previous
import jax
import jax.numpy as jnp
from jax import lax
from jax.experimental import pallas as pl
from jax.experimental.pallas import tpu as pltpu
from jax.experimental.pallas import tpu_sc as plsc

SEQ_LEN = 8192
MODEL_DIM = 1024

_info = plsc.get_sparse_core_info()
_NC, _NS = _info.num_cores, _info.num_subcores
_NW = _NC * _NS
_ROWS_PER_W = SEQ_LEN // _NW
_HALF = _ROWS_PER_W // 2
_CHUNK = 32
_NCHUNKS = _HALF // _CHUNK
_NBUF = 2


def _copy_body(table_hbm, out_hbm, bufa0, bufa1, shared,
               al0, al1, as0, as1, bl0, bl1, bs0, bs1):
    sid = lax.axis_index("s")
    wid = sid * _NC + lax.axis_index("c")
    base_a = wid * _ROWS_PER_W
    base_b = base_a + _HALF
    bufs_a = (bufa0, bufa1)
    sem_al = (al0, al1)
    sem_as = (as0, as1)
    sem_bl = (bl0, bl1)
    sem_bs = (bs0, bs1)

    def load_a(i):
        b = i % _NBUF
        return pltpu.make_async_copy(
            table_hbm.at[pl.ds(base_a + i * _CHUNK, _CHUNK), :],
            bufs_a[b], sem_al[b])

    def store_a(i):
        b = i % _NBUF
        return pltpu.make_async_copy(
            bufs_a[b], out_hbm.at[pl.ds(base_a + i * _CHUNK, _CHUNK), :],
            sem_as[b])

    def load_b(i):
        b = i % _NBUF
        return pltpu.make_async_copy(
            table_hbm.at[pl.ds(base_b + i * _CHUNK, _CHUNK), :],
            shared.at[sid, b], sem_bl[b])

    def store_b(i):
        b = i % _NBUF
        return pltpu.make_async_copy(
            shared.at[sid, b], out_hbm.at[pl.ds(base_b + i * _CHUNK, _CHUNK), :],
            sem_bs[b])

    for i in range(_NBUF):
        load_a(i).start()
        load_b(i).start()
    for i in range(_NCHUNKS):
        load_a(i).wait()
        store_a(i).start()
        load_b(i).wait()
        store_b(i).start()
        ni = i + _NBUF
        store_a(i).wait()
        store_b(i).wait()
        if ni < _NCHUNKS:
            load_a(ni).start()
            load_b(ni).start()


def kernel(x, emb_weight):
    mesh = plsc.VectorSubcoreMesh(core_axis_name="c", subcore_axis_name="s")
    copy = pl.kernel(
        _copy_body,
        mesh=mesh,
        out_type=jax.ShapeDtypeStruct((SEQ_LEN, MODEL_DIM), jnp.float32),
        scratch_types=[
            pltpu.VMEM((_CHUNK, MODEL_DIM), jnp.float32),
            pltpu.VMEM((_CHUNK, MODEL_DIM), jnp.float32),
            pltpu.VMEM_SHARED((_NS, _NBUF, _CHUNK, MODEL_DIM), jnp.float32),
            pltpu.SemaphoreType.DMA,
            pltpu.SemaphoreType.DMA,
            pltpu.SemaphoreType.DMA,
            pltpu.SemaphoreType.DMA,
            pltpu.SemaphoreType.DMA,
            pltpu.SemaphoreType.DMA,
            pltpu.SemaphoreType.DMA,
            pltpu.SemaphoreType.DMA,
        ],
    )
    return copy(emb_weight)
